# baseline (speedup 1.0000x reference)
import jax
import jax.numpy as jnp
from jax import lax
from jax.experimental import pallas as pl
from jax.experimental.pallas import tpu as pltpu
from jax.experimental.pallas import tpu_sc as plsc

B = 16384
D = 64
NC = 2
NS = 16
NW = NC * NS
BPW = B // NW
C = 64
NCH = BPW // C
NT = 11


def _sc_body(obs_0, obs_1, obs_2, obs_3, obs_4, obs_5, obs_6, obs_7,
             prev_action,
             w_state_0, w_state_1, w_state_2, w_state_3,
             w_state_4, w_state_5, w_state_6, w_state_7,
             w_act_0, w_act_1, w_act_2, bias,
             out,
             idx_buf, rows, out_buf, bias_buf,
             sem_g0, sem_g1, sem_o0, sem_o1):
    obs = (obs_0, obs_1, obs_2, obs_3, obs_4, obs_5, obs_6, obs_7)
    tables = (w_state_0, w_state_1, w_state_2, w_state_3,
              w_state_4, w_state_5, w_state_6, w_state_7,
              w_act_0, w_act_1, w_act_2)
    sem_g = (sem_g0, sem_g1)
    sem_o = (sem_o0, sem_o1)

    wid = lax.axis_index("s") * NC + lax.axis_index("c")
    obase = wid * BPW

    pltpu.sync_copy(bias, bias_buf)
    for t in range(8):
        pltpu.sync_copy(obs[t].at[pl.ds(obase, BPW)], idx_buf.at[t])
    pltpu.sync_copy(prev_action.at[pl.ds(obase, BPW)], idx_buf.at[8])

    ten = jnp.full((16,), 10, jnp.int32)
    tenth = jnp.full((16,), 0.1, jnp.float32)
    for j in range(BPW // 16):
        sl = pl.ds(j * 16, 16)
        v = idx_buf[8, sl]
        q1 = (v.astype(jnp.float32) * tenth).astype(jnp.int32)
        q2 = (q1.astype(jnp.float32) * tenth).astype(jnp.int32)
        q3 = (q2.astype(jnp.float32) * tenth).astype(jnp.int32)
        idx_buf[8, sl] = v - q1 * ten
        idx_buf[9, sl] = q1 - q2 * ten
        idx_buf[10, sl] = q2 - q3 * ten

    def fire(g, b):
        for t in range(NT):
            pltpu.async_copy(
                tables[t].at[idx_buf.at[t, pl.ds(g * C, C)]],
                rows.at[b, t], sem_g[b])

    def wait_gathers(g, b):
        for t in range(NT):
            pltpu.make_async_copy(
                tables[t].at[idx_buf.at[t, pl.ds(g * C, C)]],
                rows.at[b, t], sem_g[b]).wait()

    fire(0, 0)

    def outer(i, carry):
        for b in range(2):
            g = 2 * i + b

            @pl.when(g + 1 < NCH)
            def _():
                fire(g + 1, 1 - b)

            wait_gathers(g, b)

            @pl.when(g >= 2)
            def _():
                pltpu.make_async_copy(
                    out_buf.at[b],
                    out.at[pl.ds(obase + (g - 2) * C, C)], sem_o[b]).wait()

            def row_body(i, rcarry):
                for rr in range(4):
                    r = i * 4 + rr
                    for kcol in range(D // 16):
                        sl = pl.ds(kcol * 16, 16)
                        acc = bias_buf[sl]
                        for t in range(NT):
                            acc = acc + rows[b, t, r, sl]
                        out_buf[b, r, sl] = acc
                return rcarry

            lax.fori_loop(0, C // 4, row_body, 0)
            pltpu.async_copy(
                out_buf.at[b], out.at[pl.ds(obase + g * C, C)], sem_o[b])
        return carry

    lax.fori_loop(0, NCH // 2, outer, 0)

    for b in range(2):
        g = NCH - 2 + b
        pltpu.make_async_copy(
            out_buf.at[b], out.at[pl.ds(obase + g * C, C)], sem_o[b]).wait()


def kernel(obs_0, obs_1, obs_2, obs_3, obs_4, obs_5, obs_6, obs_7,
           prev_action,
           w_state_0, w_state_1, w_state_2, w_state_3,
           w_state_4, w_state_5, w_state_6, w_state_7,
           w_act_0, w_act_1, w_act_2, bias):
    mesh = plsc.VectorSubcoreMesh(core_axis_name="c", subcore_axis_name="s")
    run = pl.kernel(
        _sc_body,
        out_type=jax.ShapeDtypeStruct((B, D), jnp.float32),
        mesh=mesh,
        scratch_types=[
            pltpu.VMEM((NT, BPW), jnp.int32),
            pltpu.VMEM((2, NT, C, D), jnp.float32),
            pltpu.VMEM((2, C, D), jnp.float32),
            pltpu.VMEM((D,), jnp.float32),
            pltpu.SemaphoreType.DMA,
            pltpu.SemaphoreType.DMA,
            pltpu.SemaphoreType.DMA,
            pltpu.SemaphoreType.DMA,
        ],
        compiler_params=pltpu.CompilerParams(use_tc_tiling_on_sc=False),
    )
    return run(obs_0, obs_1, obs_2, obs_3, obs_4, obs_5, obs_6, obs_7,
               prev_action,
               w_state_0, w_state_1, w_state_2, w_state_3,
               w_state_4, w_state_5, w_state_6, w_state_7,
               w_act_0, w_act_1, w_act_2, bias)


# speedup vs baseline: 1.0060x; 1.0060x over previous
import jax
import jax.numpy as jnp
from jax import lax
from jax.experimental import pallas as pl
from jax.experimental.pallas import tpu as pltpu
from jax.experimental.pallas import tpu_sc as plsc

B = 16384
D = 64
NC = 2
NS = 16
NW = NC * NS
BPW = B // NW
C = 64
NCH = BPW // C
NT = 11


def _sc_body(obs_0, obs_1, obs_2, obs_3, obs_4, obs_5, obs_6, obs_7,
             prev_action,
             w_state_0, w_state_1, w_state_2, w_state_3,
             w_state_4, w_state_5, w_state_6, w_state_7,
             w_act_0, w_act_1, w_act_2, bias,
             out,
             idx_buf, rows, out_buf, bias_buf,
             sem_g0, sem_g1, sem_o0, sem_o1):
    obs = (obs_0, obs_1, obs_2, obs_3, obs_4, obs_5, obs_6, obs_7)
    tables = (w_state_0, w_state_1, w_state_2, w_state_3,
              w_state_4, w_state_5, w_state_6, w_state_7,
              w_act_0, w_act_1, w_act_2)
    sem_g = (sem_g0, sem_g1)
    sem_o = (sem_o0, sem_o1)

    wid = lax.axis_index("s") * NC + lax.axis_index("c")
    obase = wid * BPW

    pltpu.sync_copy(bias, bias_buf)
    for t in range(8):
        pltpu.sync_copy(obs[t].at[pl.ds(obase, BPW)], idx_buf.at[t])
    pltpu.sync_copy(prev_action.at[pl.ds(obase, BPW)], idx_buf.at[8])

    ten = jnp.full((16,), 10, jnp.int32)
    tenth = jnp.full((16,), 0.1, jnp.float32)
    for j in range(BPW // 16):
        sl = pl.ds(j * 16, 16)
        v = idx_buf[8, sl]
        q1 = (v.astype(jnp.float32) * tenth).astype(jnp.int32)
        q2 = (q1.astype(jnp.float32) * tenth).astype(jnp.int32)
        q3 = (q2.astype(jnp.float32) * tenth).astype(jnp.int32)
        idx_buf[8, sl] = v - q1 * ten
        idx_buf[9, sl] = q1 - q2 * ten
        idx_buf[10, sl] = q2 - q3 * ten

    def fire(g, b):
        for t in range(NT):
            pltpu.async_copy(
                tables[t].at[idx_buf.at[t, pl.ds(g * C, C)]],
                rows.at[b, t], sem_g[b])

    def wait_gathers(g, b):
        for t in range(NT):
            pltpu.make_async_copy(
                tables[t].at[idx_buf.at[t, pl.ds(g * C, C)]],
                rows.at[b, t], sem_g[b]).wait()

    fire(0, 0)

    def outer(i, carry):
        for b in range(2):
            g = 2 * i + b

            @pl.when(g + 1 < NCH)
            def _():
                fire(g + 1, 1 - b)

            wait_gathers(g, b)

            @pl.when(g >= 2)
            def _():
                pltpu.make_async_copy(
                    out_buf.at[b],
                    out.at[pl.ds(obase + (g - 2) * C, C)], sem_o[b]).wait()

            def row_body(i, rcarry):
                for rr in range(4):
                    r = i * 4 + rr
                    for kcol in range(D // 16):
                        sl = pl.ds(kcol * 16, 16)
                        vals = [rows[b, t, r, sl] for t in range(NT)]
                        vals.append(bias_buf[sl])
                        while len(vals) > 1:
                            vals = [vals[j] + vals[j + 1]
                                    for j in range(0, len(vals) - 1, 2)] + (
                                [vals[-1]] if len(vals) % 2 else [])
                        out_buf[b, r, sl] = vals[0]
                return rcarry

            lax.fori_loop(0, C // 4, row_body, 0)
            pltpu.async_copy(
                out_buf.at[b], out.at[pl.ds(obase + g * C, C)], sem_o[b])
        return carry

    lax.fori_loop(0, NCH // 2, outer, 0)

    for b in range(2):
        g = NCH - 2 + b
        pltpu.make_async_copy(
            out_buf.at[b], out.at[pl.ds(obase + g * C, C)], sem_o[b]).wait()


def kernel(obs_0, obs_1, obs_2, obs_3, obs_4, obs_5, obs_6, obs_7,
           prev_action,
           w_state_0, w_state_1, w_state_2, w_state_3,
           w_state_4, w_state_5, w_state_6, w_state_7,
           w_act_0, w_act_1, w_act_2, bias):
    mesh = plsc.VectorSubcoreMesh(core_axis_name="c", subcore_axis_name="s")
    run = pl.kernel(
        _sc_body,
        out_type=jax.ShapeDtypeStruct((B, D), jnp.float32),
        mesh=mesh,
        scratch_types=[
            pltpu.VMEM((NT, BPW), jnp.int32),
            pltpu.VMEM((2, NT, C, D), jnp.float32),
            pltpu.VMEM((2, C, D), jnp.float32),
            pltpu.VMEM((D,), jnp.float32),
            pltpu.SemaphoreType.DMA,
            pltpu.SemaphoreType.DMA,
            pltpu.SemaphoreType.DMA,
            pltpu.SemaphoreType.DMA,
        ],
        compiler_params=pltpu.CompilerParams(use_tc_tiling_on_sc=False),
    )
    return run(obs_0, obs_1, obs_2, obs_3, obs_4, obs_5, obs_6, obs_7,
               prev_action,
               w_state_0, w_state_1, w_state_2, w_state_3,
               w_state_4, w_state_5, w_state_6, w_state_7,
               w_act_0, w_act_1, w_act_2, bias)
